# baseline (speedup 1.0000x reference)
import functools

import jax
import jax.numpy as jnp
from jax.experimental import pallas as pl

_N_WAY = 5
_K_SHOT = 5
_HW = 25
_TEMP = 2.0
_NEG = -1e30


def _dmn4_kernel(a_ref, bp_ref, qy_ref, o_ref, *, qt, nq):
    bi = pl.program_id(0)
    ti = pl.program_id(1)
    f32 = jnp.float32

    @pl.when((bi == 0) & (ti == 0))
    def _init():
        o_ref[...] = jnp.zeros((1, 1), f32)

    a3 = a_ref[0]
    bcat = bp_ref[0].reshape(5 * 128, 640)

    parts = [a3[:, :, dc * _HW:(dc + 1) * _HW] for dc in range(5)]
    aall = jnp.concatenate(parts, axis=1)

    dn = (((0,), (0,)), ((), ()))
    gs = [jax.lax.dot_general(aall[i], bcat, dn, preferred_element_type=f32)
          for i in range(qt)]
    g = jnp.stack(gs, axis=0)

    t = jax.lax.dot_general(a3 * a3, jnp.ones((qt, 128, 1), f32),
                            (((1,), (1,)), ((0,), (0,))))
    qn2 = t[:, 0 * _HW:1 * _HW, :]
    for dc in range(1, 5):
        qn2 = qn2 + t[:, dc * _HW:(dc + 1) * _HW, :]
    rqn = 1.0 / jnp.maximum(jnp.sqrt(qn2), 1e-12)
    sn = jnp.maximum(jnp.sqrt(jnp.sum(bcat * bcat, axis=0, keepdims=True)), 1e-12)
    gn = g / sn[None]

    lane = jax.lax.broadcasted_iota(jnp.int32, (1, 1, 5 * 128), 2)
    rowi = jax.lax.broadcasted_iota(jnp.int32, (1, _HW, 1), 1)
    colvalid = (lane - (lane // 128) * 128) < _K_SHOT * _HW

    sm = jnp.where(colvalid, gn, _NEG)
    maxv = jnp.max(sm, axis=2, keepdims=True)
    jp = jnp.min(jnp.where(sm == maxv, lane, 5 * 128), axis=2, keepdims=True)

    cms = [jnp.max(sm[:, :, n * 128:(n + 1) * 128], axis=2, keepdims=True)
           for n in range(_N_WAY)]

    found = jnp.zeros(maxv.shape, dtype=jnp.bool_)
    second = jnp.full(maxv.shape, _NEG, dtype=f32)
    for n in range(_N_WAY):
        is_max = cms[n] == maxv
        is_first = is_max & (~found)
        found = found | is_max
        second = jnp.where(is_first, second, jnp.maximum(second, cms[n]))
    diff = (maxv - second) * rqn

    oh = lane == jp
    dm = jnp.where(oh, diff, 0.0)
    colmax = jnp.max(dm, axis=1, keepdims=True)
    wrow = jnp.min(jnp.where(dm == colmax, rowi, 1000), axis=1, keepdims=True)
    mi = jnp.max(jnp.where(oh & (wrow == rowi), 1.0, 0.0), axis=2, keepdims=True)

    logits = [jnp.sum((cms[n] * rqn) * mi, axis=1, keepdims=True) * _TEMP
              for n in range(_N_WAY)]

    qy = qy_ref[0]
    m = logits[0]
    for n in range(1, _N_WAY):
        m = jnp.maximum(m, logits[n])
    se = jnp.zeros(m.shape, f32)
    sel = jnp.zeros(m.shape, f32)
    for n in range(_N_WAY):
        se = se + jnp.exp(logits[n] - m)
        sel = sel + jnp.where(qy == n, logits[n], 0.0)
    nll = (m + jnp.log(se)) - sel
    o_ref[...] += jnp.sum(nll, axis=0) / nq


def kernel(support_xf, support_y, query_xf, query_y):
    del support_y
    b, q, c, h, w = query_xf.shape
    hw = h * w
    qt = 25
    nt = q // qt

    a = query_xf.reshape(b, q, c // 5, 5 * hw)
    bp = support_xf.reshape(b, _N_WAY, _K_SHOT, c // 5, 5, hw)
    bp = bp.transpose(0, 4, 3, 1, 2, 5)
    bp = bp.reshape(b, 5, c // 5, _N_WAY, _K_SHOT * hw)
    bp = jnp.pad(bp, ((0, 0),) * 4 + ((0, 128 - _K_SHOT * hw),))
    bp = bp.reshape(b, 5, c // 5, _N_WAY * 128)
    qy = query_y.astype(jnp.int32).reshape(b, q, 1, 1)

    out = pl.pallas_call(
        functools.partial(_dmn4_kernel, qt=qt, nq=b * q),
        grid=(b, nt),
        in_specs=[
            pl.BlockSpec((1, qt, c // 5, 5 * hw), lambda bi, ti: (bi, ti, 0, 0)),
            pl.BlockSpec((1, 5, c // 5, _N_WAY * 128), lambda bi, ti: (bi, 0, 0, 0)),
            pl.BlockSpec((1, qt, 1, 1), lambda bi, ti: (bi, ti, 0, 0)),
        ],
        out_specs=pl.BlockSpec((1, 1), lambda bi, ti: (0, 0)),
        out_shape=jax.ShapeDtypeStruct((1, 1), jnp.float32),
    )(a, bp, qy)
    return out[0, 0]


# speedup vs baseline: 2.4190x; 2.4190x over previous
import functools

import jax
import jax.numpy as jnp
from jax.experimental import pallas as pl

_N_WAY = 5
_K_SHOT = 5
_TEMP = 2.0
_NEG = -1e30


def _dmn4_kernel(a_ref, b_ref, qy_ref, o_ref, *, qt, nq):
    bi = pl.program_id(0)
    ti = pl.program_id(1)

    @pl.when((bi == 0) & (ti == 0))
    def _init():
        o_ref[...] = jnp.zeros((1, 1), jnp.float32)

    a2 = a_ref[0]
    bm = b_ref[0]

    g = jnp.dot(a2, bm, preferred_element_type=jnp.float32)
    qn = jnp.maximum(jnp.sqrt(jnp.sum(a2 * a2, axis=1, keepdims=True)), 1e-12)
    sn = jnp.maximum(jnp.sqrt(jnp.sum(bm * bm, axis=0, keepdims=True)), 1e-12)
    s3 = (g / (qn * sn)).reshape(qt, 32, 5 * 128)

    lane = jax.lax.broadcasted_iota(jnp.int32, (1, 1, 5 * 128), 2)
    rowi = jax.lax.broadcasted_iota(jnp.int32, (1, 32, 1), 1)
    colvalid = (lane - (lane // 128) * 128) < 125

    sm = jnp.where(colvalid, s3, _NEG)
    maxv = jnp.max(sm, axis=2, keepdims=True)
    jp = jnp.min(jnp.where(sm == maxv, lane, 5 * 128), axis=2, keepdims=True)

    cms = [jnp.max(sm[:, :, n * 128:(n + 1) * 128], axis=2, keepdims=True)
           for n in range(_N_WAY)]

    found = jnp.zeros(maxv.shape, dtype=jnp.bool_)
    second = jnp.full(maxv.shape, _NEG, dtype=jnp.float32)
    for n in range(_N_WAY):
        is_max = cms[n] == maxv
        is_first = is_max & (~found)
        found = found | is_max
        second = jnp.where(is_first, second, jnp.maximum(second, cms[n]))
    diff = maxv - second

    oh = lane == jp
    dm = jnp.where(oh, diff, 0.0)
    colmax = jnp.max(dm, axis=1, keepdims=True)
    wrow = jnp.min(jnp.where(dm == colmax, rowi, 1000), axis=1, keepdims=True)
    mi = jnp.max(jnp.where(oh & (wrow == rowi), 1.0, 0.0), axis=2, keepdims=True)

    logits = [jnp.sum(cms[n] * mi, axis=1, keepdims=True) * _TEMP
              for n in range(_N_WAY)]

    qy = qy_ref[0]
    m = logits[0]
    for n in range(1, _N_WAY):
        m = jnp.maximum(m, logits[n])
    se = jnp.zeros(m.shape, jnp.float32)
    sel = jnp.zeros(m.shape, jnp.float32)
    for n in range(_N_WAY):
        se = se + jnp.exp(logits[n] - m)
        sel = sel + jnp.where(qy == n, logits[n], 0.0)
    nll = (m + jnp.log(se)) - sel
    o_ref[...] += jnp.sum(nll, axis=0) / nq


def kernel(support_xf, support_y, query_xf, query_y):
    del support_y
    b, q, c, h, w = query_xf.shape
    hw = h * w
    qt = 25
    nt = q // qt

    a = query_xf.reshape(b, q, c, hw).transpose(0, 1, 3, 2)
    a = jnp.pad(a, ((0, 0), (0, 0), (0, 32 - hw), (0, 0)))
    a = a.reshape(b, q * 32, c)
    bm = support_xf.reshape(b, _N_WAY, _K_SHOT, c, hw)
    bm = bm.transpose(0, 3, 1, 2, 4).reshape(b, c, _N_WAY, _K_SHOT * hw)
    bm = jnp.pad(bm, ((0, 0), (0, 0), (0, 0), (0, 128 - _K_SHOT * hw)))
    bm = bm.reshape(b, c, _N_WAY * 128)
    qy = query_y.astype(jnp.int32).reshape(b, q, 1, 1)

    out = pl.pallas_call(
        functools.partial(_dmn4_kernel, qt=qt, nq=b * q),
        grid=(b, nt),
        in_specs=[
            pl.BlockSpec((1, qt * 32, c), lambda bi, ti: (bi, ti, 0)),
            pl.BlockSpec((1, c, _N_WAY * 128), lambda bi, ti: (bi, 0, 0)),
            pl.BlockSpec((1, qt, 1, 1), lambda bi, ti: (bi, ti, 0, 0)),
        ],
        out_specs=pl.BlockSpec((1, 1), lambda bi, ti: (0, 0)),
        out_shape=jax.ShapeDtypeStruct((1, 1), jnp.float32),
    )(a, bm, qy)
    return out[0, 0]


# speedup vs baseline: 2.7033x; 1.1175x over previous
import functools

import jax
import jax.numpy as jnp
from jax.experimental import pallas as pl
from jax.experimental.pallas import tpu as pltpu

_N_WAY = 5
_K_SHOT = 5
_TEMP = 2.0
_NEG = -1e30


def _dmn4_kernel(a_ref, b_ref, qy_ref, o_ref, a_scr, *, qt, nq):
    bi = pl.program_id(0)
    ti = pl.program_id(1)

    @pl.when((bi == 0) & (ti == 0))
    def _init():
        o_ref[...] = jnp.zeros((1, 1), jnp.float32)
        a_scr[...] = jnp.zeros(a_scr.shape, jnp.float32)

    a4 = a_ref[0]
    for i in range(qt):
        a_scr[i * 32:i * 32 + 25, :] = a4[i]
    a2 = a_scr[...]
    bm = b_ref[0]

    g = jnp.dot(a2, bm, preferred_element_type=jnp.float32)
    qn = jnp.maximum(jnp.sqrt(jnp.sum(a2 * a2, axis=1, keepdims=True)), 1e-12)
    sn = jnp.maximum(jnp.sqrt(jnp.sum(bm * bm, axis=0, keepdims=True)), 1e-12)
    s3 = (g / (qn * sn)).reshape(qt, 32, 5 * 128)

    lane = jax.lax.broadcasted_iota(jnp.int32, (1, 1, 5 * 128), 2)
    rowi = jax.lax.broadcasted_iota(jnp.int32, (1, 32, 1), 1)
    colvalid = (lane - (lane // 128) * 128) < 125

    sm = jnp.where(colvalid, s3, _NEG)
    maxv = jnp.max(sm, axis=2, keepdims=True)
    jp = jnp.min(jnp.where(sm == maxv, lane, 5 * 128), axis=2, keepdims=True)

    cms = [jnp.max(sm[:, :, n * 128:(n + 1) * 128], axis=2, keepdims=True)
           for n in range(_N_WAY)]

    found = jnp.zeros(maxv.shape, dtype=jnp.bool_)
    second = jnp.full(maxv.shape, _NEG, dtype=jnp.float32)
    for n in range(_N_WAY):
        is_max = cms[n] == maxv
        is_first = is_max & (~found)
        found = found | is_max
        second = jnp.where(is_first, second, jnp.maximum(second, cms[n]))
    diff = maxv - second

    oh = lane == jp
    dm = jnp.where(oh, diff, 0.0)
    colmax = jnp.max(dm, axis=1, keepdims=True)
    wrow = jnp.min(jnp.where(dm == colmax, rowi, 1000), axis=1, keepdims=True)
    mi = jnp.max(jnp.where(oh & (wrow == rowi), 1.0, 0.0), axis=2, keepdims=True)

    logits = [jnp.sum(cms[n] * mi, axis=1, keepdims=True) * _TEMP
              for n in range(_N_WAY)]

    qy = qy_ref[0]
    m = logits[0]
    for n in range(1, _N_WAY):
        m = jnp.maximum(m, logits[n])
    se = jnp.zeros(m.shape, jnp.float32)
    sel = jnp.zeros(m.shape, jnp.float32)
    for n in range(_N_WAY):
        se = se + jnp.exp(logits[n] - m)
        sel = sel + jnp.where(qy == n, logits[n], 0.0)
    nll = (m + jnp.log(se)) - sel
    o_ref[...] += jnp.sum(nll, axis=0) / nq


def kernel(support_xf, support_y, query_xf, query_y):
    del support_y
    b, q, c, h, w = query_xf.shape
    hw = h * w
    qt = 25
    nt = q // qt

    a = query_xf.reshape(b, q, c, hw).transpose(0, 1, 3, 2)
    bm = support_xf.reshape(b, _N_WAY, _K_SHOT, c, hw)
    bm = bm.transpose(0, 3, 1, 2, 4).reshape(b, c, _N_WAY, _K_SHOT * hw)
    bm = jnp.pad(bm, ((0, 0), (0, 0), (0, 0), (0, 128 - _K_SHOT * hw)))
    bm = bm.reshape(b, c, _N_WAY * 128)
    qy = query_y.astype(jnp.int32).reshape(b, q, 1, 1)

    out = pl.pallas_call(
        functools.partial(_dmn4_kernel, qt=qt, nq=b * q),
        grid=(b, nt),
        in_specs=[
            pl.BlockSpec((1, qt, hw, c), lambda bi, ti: (bi, ti, 0, 0)),
            pl.BlockSpec((1, c, _N_WAY * 128), lambda bi, ti: (bi, 0, 0)),
            pl.BlockSpec((1, qt, 1, 1), lambda bi, ti: (bi, ti, 0, 0)),
        ],
        out_specs=pl.BlockSpec((1, 1), lambda bi, ti: (0, 0)),
        out_shape=jax.ShapeDtypeStruct((1, 1), jnp.float32),
        scratch_shapes=[pltpu.VMEM((qt * 32, c), jnp.float32)],
    )(a, bm, qy)
    return out[0, 0]
